# baseline (speedup 1.0000x reference)
import functools

import jax
import jax.numpy as jnp
from jax import lax
from jax.experimental import pallas as pl
from jax.experimental.pallas import tpu as pltpu
from jax.experimental.pallas import tpu_sc as plsc


def kernel(h, r, t, ent_embeddings, rel_embeddings):
    B = h.shape[0]
    D = ent_embeddings.shape[1]
    info = plsc.get_sparse_core_info()
    NC, NS = info.num_cores, info.num_subcores
    NW = NC * NS
    b_per_w = B // NW
    CHUNK = 16

    mesh = plsc.VectorSubcoreMesh(core_axis_name="c", subcore_axis_name="s")
    out_t = jax.ShapeDtypeStruct((B, D), jnp.float32)

    @functools.partial(
        pl.kernel,
        mesh=mesh,
        out_type=[out_t, out_t, out_t],
        scratch_types=[
            pltpu.VMEM((b_per_w,), jnp.int32),
            pltpu.VMEM((b_per_w,), jnp.int32),
            pltpu.VMEM((b_per_w,), jnp.int32),
            pltpu.SemaphoreType.DMA,
        ],
    )
    def gather3(h_hbm, r_hbm, t_hbm, ent_hbm, rel_hbm, oh, ot, orr,
                h_v, r_v, t_v, sem):
        wid = lax.axis_index("s") * NC + lax.axis_index("c")
        base = wid * b_per_w
        pltpu.sync_copy(h_hbm.at[pl.ds(base, b_per_w)], h_v)
        pltpu.sync_copy(t_hbm.at[pl.ds(base, b_per_w)], t_v)
        pltpu.sync_copy(r_hbm.at[pl.ds(base, b_per_w)], r_v)

        def issue(i0):
            hvec = h_v[pl.ds(i0 * CHUNK, CHUNK)]
            tvec = t_v[pl.ds(i0 * CHUNK, CHUNK)]
            rvec = r_v[pl.ds(i0 * CHUNK, CHUNK)]
            for k in range(CHUNK):
                i = i0 * CHUNK + k
                pltpu.async_copy(ent_hbm.at[pl.ds(hvec[k], 1)],
                                 oh.at[pl.ds(base + i, 1)], sem)
                pltpu.async_copy(ent_hbm.at[pl.ds(tvec[k], 1)],
                                 ot.at[pl.ds(base + i, 1)], sem)
                pltpu.async_copy(rel_hbm.at[pl.ds(rvec[k], 1)],
                                 orr.at[pl.ds(base + i, 1)], sem)

        pl.loop(0, b_per_w // CHUNK)(issue)

        pltpu.make_async_copy(
            ent_hbm.at[pl.ds(0, b_per_w)], oh.at[pl.ds(base, b_per_w)], sem
        ).wait()
        pltpu.make_async_copy(
            ent_hbm.at[pl.ds(0, b_per_w)], ot.at[pl.ds(base, b_per_w)], sem
        ).wait()
        pltpu.make_async_copy(
            ent_hbm.at[pl.ds(0, b_per_w)], orr.at[pl.ds(base, b_per_w)], sem
        ).wait()

    h_e, t_e, r_e = gather3(h, r, t, ent_embeddings, rel_embeddings)
    return (h_e, t_e, r_e)


# speedup vs baseline: 2.7761x; 2.7761x over previous
import functools

import jax
import jax.numpy as jnp
from jax import lax
from jax.experimental import pallas as pl
from jax.experimental.pallas import tpu as pltpu
from jax.experimental.pallas import tpu_sc as plsc


def kernel(h, r, t, ent_embeddings, rel_embeddings):
    B = h.shape[0]
    D = ent_embeddings.shape[1]
    info = plsc.get_sparse_core_info()
    NC, NS, L = info.num_cores, info.num_subcores, info.num_lanes
    NW = NC * NS
    b_per_w = B // NW
    C = 128
    NCHUNK = b_per_w // C

    mesh = plsc.VectorSubcoreMesh(core_axis_name="c", subcore_axis_name="s")
    out_t = jax.ShapeDtypeStruct((B, D), jnp.float32)

    @functools.partial(
        pl.kernel,
        mesh=mesh,
        out_type=[out_t, out_t, out_t],
        scratch_types=[
            pltpu.VMEM((b_per_w,), jnp.int32),
            pltpu.VMEM((b_per_w,), jnp.int32),
            pltpu.VMEM((b_per_w,), jnp.int32),
            pltpu.VMEM((C, D), jnp.float32),
            pltpu.SemaphoreType.DMA,
        ],
    )
    def gather3(h_hbm, r_hbm, t_hbm, ent_hbm, rel_hbm, oh, ot, orr,
                h_v, t_v, r_v, rows_v, sem):
        wid = lax.axis_index("s") * NC + lax.axis_index("c")
        base = wid * b_per_w
        pltpu.sync_copy(h_hbm.at[pl.ds(base, b_per_w)], h_v)
        pltpu.sync_copy(t_hbm.at[pl.ds(base, b_per_w)], t_v)
        pltpu.sync_copy(r_hbm.at[pl.ds(base, b_per_w)], r_v)

        def table_pass(idx_v, table_hbm, out_hbm):
            def chunk(c0):
                off = c0 * C
                copies = []
                for g in range(C // L):
                    s = idx_v[pl.ds(off + g * L, L)]
                    for k in range(L):
                        item = g * L + k
                        copies.append(pltpu.async_copy(
                            table_hbm.at[pl.ds(s[k], 1)],
                            rows_v.at[pl.ds(item, 1)], sem))
                for cp in copies:
                    cp.wait()
                pltpu.sync_copy(rows_v, out_hbm.at[pl.ds(base + off, C)])

            pl.loop(0, NCHUNK)(chunk)

        table_pass(h_v, ent_hbm, oh)
        table_pass(t_v, ent_hbm, ot)
        table_pass(r_v, rel_hbm, orr)

    h_e, t_e, r_e = gather3(h, r, t, ent_embeddings, rel_embeddings)
    return (h_e, t_e, r_e)
